# baseline (speedup 1.0000x reference)
import jax
import jax.numpy as jnp
from jax.experimental import pallas as pl

_BANK = 16
_BATCH = 32
_BLK = 8192


def _combine_body(idx_ref, prob_ref, p_ref, o_ref):
    idx = idx_ref[...]
    prob = prob_ref[...]
    e = jax.lax.broadcasted_iota(jnp.int32, (1, 1, _BANK), 2)
    onehot = (idx[:, :, None] == e).astype(jnp.float32)
    w = jnp.sum(prob[:, :, None] * onehot, axis=1)
    o_ref[...] = jax.lax.dot_general(
        w, p_ref[...], (((1,), (1,)), ((), ())),
        preferred_element_type=jnp.float32)


def kernel(parameter, selection_index, selection_probabilities):
    h, w_dim, bank = parameter.shape
    m = h * w_dim
    pf = parameter.reshape(m, bank)
    out = pl.pallas_call(
        _combine_body,
        grid=(m // _BLK,),
        in_specs=[
            pl.BlockSpec((_BATCH, 2), lambda i: (0, 0)),
            pl.BlockSpec((_BATCH, 2), lambda i: (0, 0)),
            pl.BlockSpec((_BLK, bank), lambda i: (i, 0)),
        ],
        out_specs=pl.BlockSpec((_BATCH, _BLK), lambda i: (0, i)),
        out_shape=jax.ShapeDtypeStruct((_BATCH, m), jnp.float32),
    )(selection_index, selection_probabilities, pf)
    return out.reshape(_BATCH, h, w_dim)


# speedup vs baseline: 1.0002x; 1.0002x over previous
import jax
import jax.numpy as jnp
from jax.experimental import pallas as pl

_BANK = 16
_BATCH = 32
_ROWS = 64
_BLK = _ROWS * 128


def _combine_body(idx_ref, prob_ref, p_ref, o_ref):
    idx = idx_ref[...]
    prob = prob_ref[...]
    e = jax.lax.broadcasted_iota(jnp.int32, (1, 1, _BANK), 2)
    onehot = (idx[:, :, None] == e).astype(jnp.float32)
    w = jnp.sum(prob[:, :, None] * onehot, axis=1)
    o_ref[...] = jax.lax.dot_general(
        w, p_ref[...], (((1,), (1,)), ((), ())),
        preferred_element_type=jnp.float32)


def kernel(parameter, selection_index, selection_probabilities):
    h, w_dim, bank = parameter.shape
    m = h * w_dim
    pf = parameter.reshape(m, bank)
    out = pl.pallas_call(
        _combine_body,
        grid=(m // _BLK,),
        in_specs=[
            pl.BlockSpec((_BATCH, 2), lambda i: (0, 0)),
            pl.BlockSpec((_BATCH, 2), lambda i: (0, 0)),
            pl.BlockSpec((_BLK, bank), lambda i: (i, 0)),
        ],
        out_specs=pl.BlockSpec((_BATCH, _BLK), lambda i: (0, i)),
        out_shape=jax.ShapeDtypeStruct((_BATCH, m), jnp.float32),
    )(selection_index, selection_probabilities, pf)
    return out.reshape(_BATCH, h, w_dim)
